# baseline (speedup 1.0000x reference)
import jax
import jax.numpy as jnp
from jax import lax
from jax.experimental import pallas as pl
from jax.experimental.pallas import tpu as pltpu
from jax.experimental.pallas import tpu_sc as plsc

NC = 2
NS = 16
NW = NC * NS
L = 16

B, N, K = 8, 49152, 4
TN = N // 128
Q = NW // B
TB = TN // Q
CC = 128 // L
NBUF = 3
GB = 8
NG = TB // GB


def _sc_body(pred_hbm, delta_hbm, tags_hbm, out_hbm,
             pred_v, delta_v, tags_v, acc_v,
             sem0, sem1, sem2):
    c = lax.axis_index("c")
    s = lax.axis_index("s")
    wid = s * NC + c
    b = wid // Q
    tn0 = (wid % Q) * TB

    sems = (sem0, sem1, sem2)

    def issue(g, buf):
        tn = tn0 + g * GB
        pltpu.async_copy(pred_hbm.at[b, pl.ds(tn, GB)],
                         pred_v.at[buf], sems[buf])
        pltpu.async_copy(delta_hbm.at[b, pl.ds(tn, GB)],
                         delta_v.at[buf], sems[buf])
        pltpu.async_copy(tags_hbm.at[pl.ds(tn, GB), pl.ds(b, 1)],
                         tags_v.at[buf], sems[buf])

    def drain(buf):
        pltpu.make_async_copy(pred_hbm.at[b, pl.ds(tn0, GB)],
                              pred_v.at[buf], sems[buf]).wait()
        pltpu.make_async_copy(delta_hbm.at[b, pl.ds(tn0, GB)],
                              delta_v.at[buf], sems[buf]).wait()
        pltpu.make_async_copy(tags_hbm.at[pl.ds(tn0, GB), pl.ds(b, 1)],
                              tags_v.at[buf], sems[buf]).wait()

    zeros = jnp.zeros((L,), jnp.float32)

    def block(pv, dv, tv, j, carry):
        accs = list(carry[:8])
        dens = list(carry[8:])
        for cc in range(CC):
            m = tv[j, 0, pl.ds(cc * L, L)].astype(jnp.float32)
            dens[cc % 4] = dens[cc % 4] + m
            ws = []
            for k in range(K):
                p = pv[j, k, pl.ds(cc * L, L)]
                d = dv[j, k, pl.ds(cc * L, L)]
                a = jnp.abs(p - d)
                t = jnp.minimum(a, 1.0)
                ws.append(t * (a + a - t))
            accs[cc % 8] = accs[cc % 8] + m * ((ws[0] + ws[1]) + (ws[2] + ws[3]))
        return tuple(accs) + tuple(dens)

    issue(0, 0)
    issue(1, 1)

    def outer(g, carry):
        buf = g % NBUF

        @pl.when((g + 2 < NG) & (buf == 1))
        def _prefetch0():
            issue(g + 2, 0)

        @pl.when((g + 2 < NG) & (buf == 2))
        def _prefetch1():
            issue(g + 2, 1)

        @pl.when((g + 2 < NG) & (buf == 0))
        def _prefetch2():
            issue(g + 2, 2)

        @pl.when(buf == 0)
        def _drain0():
            drain(0)

        @pl.when(buf == 1)
        def _drain1():
            drain(1)

        @pl.when(buf == 2)
        def _drain2():
            drain(2)

        pv = pred_v.at[buf]
        dv = delta_v.at[buf]
        tv = tags_v.at[buf]
        return lax.fori_loop(
            0, GB, lambda j, cr: block(pv, dv, tv, j, cr), carry)

    parts = lax.fori_loop(0, NG, outer, (zeros,) * 12)
    acc_loss = ((parts[0] + parts[1]) + (parts[2] + parts[3])) + (
        (parts[4] + parts[5]) + (parts[6] + parts[7]))
    aden = (parts[8] + parts[9]) + (parts[10] + parts[11])

    acc_v[0, pl.ds(0, L)] = acc_loss
    acc_v[1, pl.ds(0, L)] = aden
    pltpu.sync_copy(acc_v, out_hbm.at[wid])


@jax.jit
def _sc_loss(pred, delta, tags):
    mesh = plsc.VectorSubcoreMesh(core_axis_name="c", subcore_axis_name="s")
    f = pl.kernel(
        _sc_body,
        mesh=mesh,
        compiler_params=pltpu.CompilerParams(needs_layout_passes=False),
        out_type=jax.ShapeDtypeStruct((NW, 2, L), jnp.float32),
        scratch_types=[
            pltpu.VMEM((NBUF, GB, K, 128), jnp.float32),
            pltpu.VMEM((NBUF, GB, K, 128), jnp.float32),
            pltpu.VMEM((NBUF, GB, 1, 128), jnp.int32),
            pltpu.VMEM((2, L), jnp.float32),
            pltpu.SemaphoreType.DMA,
            pltpu.SemaphoreType.DMA,
            pltpu.SemaphoreType.DMA,
        ],
    )
    return f(pred, delta, tags)


def kernel(predict_deltas, deltas, anchors_tag):
    pv = predict_deltas.reshape(B, TN, 128, K).transpose(0, 1, 3, 2)
    dv = deltas.reshape(B, TN, 128, K).transpose(0, 1, 3, 2)
    tv = anchors_tag.reshape(B, TN, 128).transpose(1, 0, 2)
    part = _sc_loss(pv, dv, tv)
    total = jnp.sum(part[:, 0, :])
    den = jnp.sum(part[:, 1, :])
    return jnp.where(den > 0, total / (8.0 * den), jnp.float32(0.0))


# speedup vs baseline: 1.0479x; 1.0479x over previous
import jax
import jax.numpy as jnp
from jax import lax
from jax.experimental import pallas as pl
from jax.experimental.pallas import tpu as pltpu
from jax.experimental.pallas import tpu_sc as plsc

NC = 2
NS = 16
NW = NC * NS
L = 16

B, N, K = 8, 49152, 4
TN = N // 128
Q = NW // B
TB = TN // Q
CC = 128 // L
NBUF = 3
GB = 8
NG = TB // GB


def _sc_body(pred_hbm, delta_hbm, tags_hbm, out_hbm,
             pred_v, delta_v, tags_v, acc_v,
             sem0, sem1, sem2):
    c = lax.axis_index("c")
    s = lax.axis_index("s")
    wid = s * NC + c
    b = wid // Q
    tn0 = (wid % Q) * TB

    sems = (sem0, sem1, sem2)

    def issue(g, buf):
        tn = tn0 + g * GB
        pltpu.async_copy(pred_hbm.at[b, pl.ds(tn, GB)],
                         pred_v.at[buf], sems[buf])
        pltpu.async_copy(delta_hbm.at[b, pl.ds(tn, GB)],
                         delta_v.at[buf], sems[buf])
        pltpu.async_copy(tags_hbm.at[pl.ds(tn, GB), pl.ds(b, 1)],
                         tags_v.at[buf], sems[buf])

    def drain(buf):
        pltpu.make_async_copy(pred_hbm.at[b, pl.ds(tn0, GB)],
                              pred_v.at[buf], sems[buf]).wait()
        pltpu.make_async_copy(delta_hbm.at[b, pl.ds(tn0, GB)],
                              delta_v.at[buf], sems[buf]).wait()
        pltpu.make_async_copy(tags_hbm.at[pl.ds(tn0, GB), pl.ds(b, 1)],
                              tags_v.at[buf], sems[buf]).wait()

    zeros = jnp.zeros((L,), jnp.float32)

    def block(pv, dv, tv, j, carry):
        accs = list(carry[:4])
        dens = list(carry[4:])
        for cc in range(CC):
            m = tv[j, 0, pl.ds(cc * L, L)].astype(jnp.float32)
            dens[cc % 2] = dens[cc % 2] + m
            ws = []
            for k in range(K):
                p = pv[j, k, pl.ds(cc * L, L)]
                d = dv[j, k, pl.ds(cc * L, L)]
                a = jnp.abs(p - d)
                t = jnp.minimum(a, 1.0)
                ws.append(t * (a + a - t))
            accs[cc % 4] = accs[cc % 4] + m * ((ws[0] + ws[1]) + (ws[2] + ws[3]))
        return tuple(accs) + tuple(dens)

    issue(0, 0)
    issue(1, 1)

    def outer(g, carry):
        buf = g % NBUF

        @pl.when((g + 2 < NG) & (buf == 1))
        def _prefetch0():
            issue(g + 2, 0)

        @pl.when((g + 2 < NG) & (buf == 2))
        def _prefetch1():
            issue(g + 2, 1)

        @pl.when((g + 2 < NG) & (buf == 0))
        def _prefetch2():
            issue(g + 2, 2)

        @pl.when(buf == 0)
        def _drain0():
            drain(0)

        @pl.when(buf == 1)
        def _drain1():
            drain(1)

        @pl.when(buf == 2)
        def _drain2():
            drain(2)

        pv = pred_v.at[buf]
        dv = delta_v.at[buf]
        tv = tags_v.at[buf]
        return lax.fori_loop(
            0, GB, lambda j, cr: block(pv, dv, tv, j, cr), carry)

    parts = lax.fori_loop(0, NG, outer, (zeros,) * 6)
    acc_loss = (parts[0] + parts[1]) + (parts[2] + parts[3])
    aden = parts[4] + parts[5]

    acc_v[0, pl.ds(0, L)] = acc_loss
    acc_v[1, pl.ds(0, L)] = aden
    pltpu.sync_copy(acc_v, out_hbm.at[wid])


@jax.jit
def _sc_loss(pred, delta, tags):
    mesh = plsc.VectorSubcoreMesh(core_axis_name="c", subcore_axis_name="s")
    f = pl.kernel(
        _sc_body,
        mesh=mesh,
        compiler_params=pltpu.CompilerParams(needs_layout_passes=False),
        out_type=jax.ShapeDtypeStruct((NW, 2, L), jnp.float32),
        scratch_types=[
            pltpu.VMEM((NBUF, GB, K, 128), jnp.float32),
            pltpu.VMEM((NBUF, GB, K, 128), jnp.float32),
            pltpu.VMEM((NBUF, GB, 1, 128), jnp.int32),
            pltpu.VMEM((2, L), jnp.float32),
            pltpu.SemaphoreType.DMA,
            pltpu.SemaphoreType.DMA,
            pltpu.SemaphoreType.DMA,
        ],
    )
    return f(pred, delta, tags)


def kernel(predict_deltas, deltas, anchors_tag):
    pv = predict_deltas.reshape(B, TN, 128, K).transpose(0, 1, 3, 2)
    dv = deltas.reshape(B, TN, 128, K).transpose(0, 1, 3, 2)
    tv = anchors_tag.reshape(B, TN, 128).transpose(1, 0, 2)
    part = _sc_loss(pv, dv, tv)
    total = jnp.sum(part[:, 0, :])
    den = jnp.sum(part[:, 1, :])
    return jnp.where(den > 0, total / (8.0 * den), jnp.float32(0.0))
